# Initial kernel scaffold; baseline (speedup 1.0000x reference)
import jax
import jax.numpy as jnp
from jax.experimental import pallas as pl


def kernel(token_ids, weight):
    raise NotImplementedError("write your pallas kernel here")


# speedup vs baseline: 9.1648x; 9.1648x over previous
import functools

import jax
import jax.numpy as jnp
from jax import lax
from jax.experimental import pallas as pl
from jax.experimental.pallas import tpu as pltpu
from jax.experimental.pallas import tpu_sc as plsc

NUM_TOKENS = 4096 * 200
DIM = 128

_CHUNK = 128
_NBUF = 4


def _build():
    info = plsc.get_sparse_core_info()
    nw = info.num_cores * info.num_subcores
    rows_per_w = NUM_TOKENS // nw
    n_chunks = rows_per_w // _CHUNK
    n_groups = n_chunks // _NBUF
    idx_rows_per_w = n_chunks

    mesh = plsc.VectorSubcoreMesh(core_axis_name="c", subcore_axis_name="s")

    @functools.partial(
        pl.kernel,
        mesh=mesh,
        out_type=jax.ShapeDtypeStruct((NUM_TOKENS, DIM), jnp.float32),
        scratch_types=[
            pltpu.VMEM((idx_rows_per_w, _CHUNK), jnp.int32),
            pltpu.VMEM((_NBUF, _CHUNK, DIM), jnp.float32),
            pltpu.SemaphoreType.DMA,
            pltpu.SemaphoreType.DMA,
        ],
    )
    def emb(idx_hbm, table_hbm, out_hbm, idx_v, rows_v, gsem, psem):
        wid = lax.axis_index("s") * info.num_cores + lax.axis_index("c")
        base = wid * rows_per_w

        pltpu.sync_copy(idx_hbm.at[pl.ds(wid * idx_rows_per_w, idx_rows_per_w)],
                        idx_v)

        def group(g, _):
            j0 = g * _NBUF
            gets = []
            for b in range(_NBUF):
                gets.append(pltpu.async_copy(
                    table_hbm.at[idx_v.at[j0 + b]], rows_v.at[b], gsem))
            puts = []
            for b in range(_NBUF):
                gets[b].wait()
                puts.append(pltpu.async_copy(
                    rows_v.at[b],
                    out_hbm.at[pl.ds(base + (j0 + b) * _CHUNK, _CHUNK)],
                    psem))
            for b in range(_NBUF):
                puts[b].wait()
            return _

        lax.fori_loop(0, n_groups, group, None)

    return emb


_EMB = _build()


@jax.jit
def kernel(token_ids, weight):
    idx2d = token_ids.reshape(NUM_TOKENS // _CHUNK, _CHUNK).astype(jnp.int32)
    out = _EMB(idx2d, weight)
    return out.reshape(*token_ids.shape, DIM)
